# Initial kernel scaffold; baseline (speedup 1.0000x reference)
import jax
import jax.numpy as jnp
from jax.experimental import pallas as pl


def kernel(inputs, embeddings):
    raise NotImplementedError("write your pallas kernel here")


# speedup vs baseline: 9.2322x; 9.2322x over previous
import functools

import jax
import jax.numpy as jnp
from jax import lax
from jax.experimental import pallas as pl
from jax.experimental.pallas import tpu as pltpu
from jax.experimental.pallas import tpu_sc as plsc

D = 128
C = 128
NC = 2
NS = 16
NW = NC * NS


def _make_gather(V, B):
    assert B % (NW * C) == 0
    nchunk = B // (NW * C)
    b_per_w = B // NW
    mesh = plsc.VectorSubcoreMesh(core_axis_name="c", subcore_axis_name="s")

    @functools.partial(
        pl.kernel,
        out_type=jax.ShapeDtypeStruct((B, D), jnp.float32),
        mesh=mesh,
        scratch_types=[
            pltpu.VMEM((nchunk, C), jnp.int32),
            pltpu.VMEM((C, D), jnp.float32),
            pltpu.SemaphoreType.DMA,
        ],
    )
    def gather_kernel(table_hbm, idx_hbm, out_hbm, idx_v, rows_v, sem):
        wid = lax.axis_index("s") * NC + lax.axis_index("c")
        pltpu.sync_copy(idx_hbm.at[wid], idx_v)
        base = wid * b_per_w

        @pl.loop(0, nchunk)
        def _chunk(i):
            pltpu.async_copy(table_hbm.at[idx_v.at[i]], rows_v, sem).wait()
            pltpu.sync_copy(rows_v, out_hbm.at[pl.ds(base + i * C, C)])

    return gather_kernel


def kernel(inputs, embeddings):
    batch, seq = inputs.shape
    V = embeddings.shape[0]
    B = batch * seq
    idx = inputs.astype(jnp.int32).reshape(NW, B // (NW * C), C)
    table = embeddings.reshape(V, D)
    out = _make_gather(V, B)(table, idx)
    return out.reshape(batch, seq, embeddings.shape[1], embeddings.shape[2])


# speedup vs baseline: 9.4687x; 1.0256x over previous
import functools

import jax
import jax.numpy as jnp
from jax import lax
from jax.experimental import pallas as pl
from jax.experimental.pallas import tpu as pltpu
from jax.experimental.pallas import tpu_sc as plsc

D = 128
C = 128
NC = 2
NS = 16
NW = NC * NS
NBUF = 4


def _make_gather(V, B):
    assert B % (NW * C) == 0
    nchunk = B // (NW * C)
    b_per_w = B // NW
    mesh = plsc.VectorSubcoreMesh(core_axis_name="c", subcore_axis_name="s")

    @functools.partial(
        pl.kernel,
        out_type=jax.ShapeDtypeStruct((B, D), jnp.float32),
        mesh=mesh,
        scratch_types=[
            pltpu.VMEM((nchunk, C), jnp.int32),
            pltpu.VMEM((NBUF, C, D), jnp.float32),
            pltpu.SemaphoreType.DMA((NBUF,)),
            pltpu.SemaphoreType.DMA((NBUF,)),
        ],
    )
    def gather_kernel(table_hbm, idx_hbm, out_hbm, idx_v, rows_v, gsem, wsem):
        wid = lax.axis_index("s") * NC + lax.axis_index("c")
        pltpu.sync_copy(idx_hbm.at[wid], idx_v)
        base = wid * b_per_w

        def gather(i, slot):
            return pltpu.make_async_copy(
                table_hbm.at[idx_v.at[i]], rows_v.at[slot], gsem.at[slot])

        def writeback(i, slot):
            return pltpu.make_async_copy(
                rows_v.at[slot], out_hbm.at[pl.ds(base + i * C, C)],
                wsem.at[slot])

        for b in range(NBUF - 1):
            gather(b, b).start()

        @pl.loop(0, nchunk)
        def _chunk(i):
            slot = lax.rem(i, NBUF)
            ahead = i + NBUF - 1
            aslot = lax.rem(ahead, NBUF)

            @pl.when(i >= 1)
            def _():
                writeback(i - 1, aslot).wait()

            @pl.when(ahead < nchunk)
            def _():
                gather(ahead, aslot).start()

            gather(i, slot).wait()
            writeback(i, slot).start()

        writeback(nchunk - 1, (nchunk - 1) % NBUF).wait()

    return gather_kernel


def kernel(inputs, embeddings):
    batch, seq = inputs.shape
    V = embeddings.shape[0]
    B = batch * seq
    idx = inputs.astype(jnp.int32).reshape(NW, B // (NW * C), C)
    table = embeddings.reshape(V, D)
    out = _make_gather(V, B)(table, idx)
    return out.reshape(batch, seq, embeddings.shape[1], embeddings.shape[2])


# speedup vs baseline: 36.7329x; 3.8794x over previous
import functools

import jax
import jax.numpy as jnp
from jax import lax
from jax.experimental import pallas as pl
from jax.experimental.pallas import tpu as pltpu
from jax.experimental.pallas import tpu_sc as plsc

D = 128
C = 128
NC = 2
NS = 16
NW = NC * NS
NG = 3
NT = 2
L = 16


def _make_lookup(V, S, B):
    assert B == NW * C
    mesh = plsc.VectorSubcoreMesh(core_axis_name="c", subcore_axis_name="s")

    @functools.partial(
        pl.kernel,
        out_type=jax.ShapeDtypeStruct((S, D, B), jnp.float32),
        mesh=mesh,
        compiler_params=pltpu.CompilerParams(needs_layout_passes=False),
        scratch_types=[
            pltpu.VMEM((S, C), jnp.int32),
            pltpu.VMEM((NG, C, D), jnp.float32),
            pltpu.VMEM((NT, D, C), jnp.float32),
            pltpu.SemaphoreType.DMA((NG,)),
            pltpu.SemaphoreType.DMA((NT,)),
        ],
    )
    def lookup_kernel(table_hbm, idxt_hbm, out_hbm, idx_v, bufs, bufts,
                      gsem, wsem):
        wid = lax.axis_index("s") * NC + lax.axis_index("c")
        b0 = wid * C
        pltpu.sync_copy(idxt_hbm.at[:, pl.ds(b0, C)], idx_v)

        liota = lax.iota(jnp.int32, L)
        rems = [jnp.bitwise_and(liota + k, L - 1) for k in range(L)]

        def gather(s, g):
            return pltpu.make_async_copy(
                table_hbm.at[idx_v.at[s]], bufs.at[g], gsem.at[g])

        def writeback(s, t):
            return pltpu.make_async_copy(
                bufts.at[t], out_hbm.at[s, :, pl.ds(b0, C)], wsem.at[t])

        def transpose(g, t):
            src = bufs.at[g]
            dst = bufts.at[t]

            @pl.loop(0, (C // L) * (D // L))
            def _tile(q):
                rbase = (q % (C // L)) * L
                cbase = (q // (C // L)) * L
                rows = rbase + liota
                for k in range(L):
                    cols = cbase + rems[k]
                    vals = plsc.load_gather(src, [rows, cols])
                    plsc.store_scatter(dst, [cols, rows], vals)

        gather(0, 0).start()
        gather(1, 1).start()

        @pl.loop(0, S)
        def _s(s):
            g = lax.rem(s, NG)
            t = lax.rem(s, NT)

            @pl.when(s >= NT)
            def _():
                writeback(s - NT, t).wait()

            gather(s, g).wait()
            transpose(g, t)
            writeback(s, t).start()

            @pl.when(s + NG - 1 < S)
            def _():
                gather(s + NG - 1, lax.rem(s + NG - 1, NG)).start()

        writeback(S - NT, S % NT).wait()
        writeback(S - 1, (S - 1) % NT).wait()

    return lookup_kernel


def kernel(inputs, embeddings):
    batch, seq = inputs.shape
    V, W, H = embeddings.shape
    table = embeddings.reshape(V, W * H)
    idx_t = jnp.transpose(inputs.astype(jnp.int32))
    out = _make_lookup(V, seq, batch)(table, idx_t)
    out4 = out.reshape(seq, W, H, batch)
    return jnp.transpose(out4, (3, 0, 1, 2))


# speedup vs baseline: 57.9313x; 1.5771x over previous
import functools

import jax
import jax.numpy as jnp
from jax import lax
from jax.experimental import pallas as pl
from jax.experimental.pallas import tpu as pltpu
from jax.experimental.pallas import tpu_sc as plsc

D = 128
C = 128
NC = 2
NS = 16
NW = NC * NS
NG = 3
NT = 2
L = 16


def _make_lookup(V, S, B):
    assert B == NW * C
    mesh = plsc.VectorSubcoreMesh(core_axis_name="c", subcore_axis_name="s")

    @functools.partial(
        pl.kernel,
        out_type=jax.ShapeDtypeStruct((S, D, B), jnp.float32),
        mesh=mesh,
        compiler_params=pltpu.CompilerParams(needs_layout_passes=False),
        scratch_types=[
            pltpu.VMEM((S, C), jnp.int32),
            pltpu.VMEM((NG, C, D), jnp.float32),
            pltpu.VMEM((NT, D, C), jnp.float32),
            pltpu.SemaphoreType.DMA((NG,)),
            pltpu.SemaphoreType.DMA((NT,)),
        ],
    )
    def lookup_kernel(table_hbm, idxt_hbm, out_hbm, idx_v, bufs, bufts,
                      gsem, wsem):
        wid = lax.axis_index("s") * NC + lax.axis_index("c")
        b0 = wid * C
        pltpu.sync_copy(idxt_hbm.at[:, pl.ds(b0, C)], idx_v)


        def gather(s, g):
            return pltpu.make_async_copy(
                table_hbm.at[idx_v.at[s]], bufs.at[g], gsem.at[g])

        def writeback(s, t):
            return pltpu.make_async_copy(
                bufts.at[t], out_hbm.at[s, :, pl.ds(b0, C)], wsem.at[t])

        def transpose(g, t):
            src = bufs.at[g]
            dst = bufts.at[t]

            @plsc.parallel_loop(0, (D // L) * (C // L), unroll=1)
            def _blk(q):
                db = q // (C // L)
                bb = lax.rem(q, C // L)
                liota = lax.iota(jnp.int32, L)
                cols = [db * L + jnp.bitwise_and(liota + k, L - 1)
                        for k in range(L)]
                rows = bb * L + liota
                vals = [plsc.load_gather(src, [rows, cols[k]])
                        for k in range(L)]
                for k in range(L):
                    plsc.store_scatter(dst, [cols[k], rows], vals[k])

        gather(0, 0).start()
        gather(1, 1).start()

        @pl.loop(0, S)
        def _s(s):
            g = lax.rem(s, NG)
            t = lax.rem(s, NT)

            @pl.when(s >= NT)
            def _():
                writeback(s - NT, t).wait()

            gather(s, g).wait()
            transpose(g, t)
            writeback(s, t).start()

            @pl.when(s + NG - 1 < S)
            def _():
                gather(s + NG - 1, lax.rem(s + NG - 1, NG)).start()

        writeback(S - NT, S % NT).wait()
        writeback(S - 1, (S - 1) % NT).wait()

    return lookup_kernel


def kernel(inputs, embeddings):
    batch, seq = inputs.shape
    V, W, H = embeddings.shape
    table = embeddings.reshape(V, W * H)
    idx_t = jnp.transpose(inputs.astype(jnp.int32))
    out = _make_lookup(V, seq, batch)(table, idx_t)
    out4 = out.reshape(seq, W, H, batch)
    return jnp.transpose(out4, (3, 0, 1, 2))


# speedup vs baseline: 60.7351x; 1.0484x over previous
import functools

import jax
import jax.numpy as jnp
from jax import lax
from jax.experimental import pallas as pl
from jax.experimental.pallas import tpu as pltpu
from jax.experimental.pallas import tpu_sc as plsc

D = 128
C = 128
NC = 2
NS = 16
NW = NC * NS
NG = 4
NT = 3
L = 16


def _make_lookup(V, S, B):
    assert B == NW * C
    mesh = plsc.VectorSubcoreMesh(core_axis_name="c", subcore_axis_name="s")

    @functools.partial(
        pl.kernel,
        out_type=jax.ShapeDtypeStruct((S, D, B), jnp.float32),
        mesh=mesh,
        compiler_params=pltpu.CompilerParams(needs_layout_passes=False),
        scratch_types=[
            pltpu.VMEM((S, C), jnp.int32),
            pltpu.VMEM((NG, C, D), jnp.float32),
            pltpu.VMEM((NT, D, C), jnp.float32),
            pltpu.SemaphoreType.DMA((NG,)),
            pltpu.SemaphoreType.DMA((NT,)),
        ],
    )
    def lookup_kernel(table_hbm, idxt_hbm, out_hbm, idx_v, bufs, bufts,
                      gsem, wsem):
        wid = lax.axis_index("s") * NC + lax.axis_index("c")
        b0 = wid * C
        pltpu.sync_copy(idxt_hbm.at[:, pl.ds(b0, C)], idx_v)


        def gather(s, g):
            return pltpu.make_async_copy(
                table_hbm.at[idx_v.at[s]], bufs.at[g], gsem.at[g])

        def writeback(s, t):
            return pltpu.make_async_copy(
                bufts.at[t], out_hbm.at[s, :, pl.ds(b0, C)], wsem.at[t])

        def transpose(g, t):
            src = bufs.at[g]
            dst = bufts.at[t]

            @plsc.parallel_loop(0, (D // L) * (C // L), unroll=1)
            def _blk(q):
                db = q // (C // L)
                bb = lax.rem(q, C // L)
                liota = lax.iota(jnp.int32, L)
                cols = [db * L + jnp.bitwise_and(liota + k, L - 1)
                        for k in range(L)]
                rows = bb * L + liota
                vals = [plsc.load_gather(src, [rows, cols[k]])
                        for k in range(L)]
                for k in range(L):
                    plsc.store_scatter(dst, [cols[k], rows], vals[k])

        gather(0, 0).start()
        gather(1, 1).start()
        gather(2, 2).start()

        @pl.loop(0, S)
        def _s(s):
            g = lax.rem(s, NG)
            t = lax.rem(s, NT)

            @pl.when(s >= NT)
            def _():
                writeback(s - NT, t).wait()

            gather(s, g).wait()
            transpose(g, t)
            writeback(s, t).start()

            @pl.when(s + NG - 1 < S)
            def _():
                gather(s + NG - 1, lax.rem(s + NG - 1, NG)).start()

        writeback(S - NT, S % NT).wait()
        writeback(S - 1, (S - 1) % NT).wait()

    return lookup_kernel


def kernel(inputs, embeddings):
    batch, seq = inputs.shape
    V, W, H = embeddings.shape
    table = embeddings.reshape(V, W * H)
    idx_t = jnp.transpose(inputs.astype(jnp.int32))
    out = _make_lookup(V, seq, batch)(table, idx_t)
    out4 = out.reshape(seq, W, H, batch)
    return jnp.transpose(out4, (3, 0, 1, 2))
